# baseline (speedup 1.0000x reference)
import functools

import jax
import jax.numpy as jnp
from jax import lax
from jax.experimental import pallas as pl
from jax.experimental.pallas import tpu as pltpu
from jax.experimental.pallas import tpu_sc as plsc

L = 8
NT = 1024
D = 1024
N = 4
S = 2048
ROWS = N * S
NC = 2
NS = 16
NW = NC * NS
RPW = ROWS // NW
B = 4
NB = RPW // B
LANES = 16
UNROLL = 4


def _fire(w_hbm, idx_v, gath_v, sem, b, buf):
    for l in range(L):
        pltpu.async_copy(
            w_hbm.at[idx_v.at[l, pl.ds(b * B, B)]],
            gath_v.at[buf, l],
            sem,
        )


def _drain(w_hbm, idx_v, gath_v, sem, buf):
    for l in range(L):
        pltpu.make_async_copy(
            w_hbm.at[idx_v.at[l, pl.ds(0, B)]],
            gath_v.at[buf, l],
            sem,
        ).wait()


def _accum(gath_v, outb_v, buf):
    for j in range(B):
        def cbody(c, _, j=j):
            base = pl.multiple_of(c * (UNROLL * LANES), UNROLL * LANES)
            for u in range(UNROLL):
                o = pl.ds(base + u * LANES, LANES)
                acc = gath_v[buf, 0, j, o]
                for l in range(1, L):
                    acc = acc + gath_v[buf, l, j, o]
                outb_v[buf, j, o] = acc
            return 0
        lax.fori_loop(0, D // (UNROLL * LANES), cbody, 0)


def _body(x_hbm, w_hbm, out_hbm, idx_v, gath_v, outb_v, sem0, sem1):
    cid = lax.axis_index("c")
    sid = lax.axis_index("s")
    wid = sid * NC + cid
    n = wid // (S // RPW)
    s0 = (wid % (S // RPW)) * RPW
    row0 = wid * RPW

    for l in range(L):
        pltpu.sync_copy(x_hbm.at[n, l, pl.ds(s0, RPW)], idx_v.at[l])

    def off_body(i, _):
        o = pl.ds(pl.multiple_of(i * LANES, LANES), LANES)
        for l in range(1, L):
            idx_v[l, o] = idx_v[l, o] + (l * NT)
        return 0
    lax.fori_loop(0, RPW // LANES, off_body, 0)

    def _store(b, buf):
        pltpu.sync_copy(outb_v.at[buf], out_hbm.at[pl.ds(row0 + b * B, B)])

    _fire(w_hbm, idx_v, gath_v, sem0, 0, 0)

    def outer(bb, _):
        b0 = 2 * bb
        b1 = 2 * bb + 1
        _fire(w_hbm, idx_v, gath_v, sem1, b1, 1)
        _drain(w_hbm, idx_v, gath_v, sem0, 0)
        _accum(gath_v, outb_v, 0)
        _store(b0, 0)
        _fire(w_hbm, idx_v, gath_v, sem0, jnp.minimum(b1 + 1, NB - 1), 0)
        _drain(w_hbm, idx_v, gath_v, sem1, 1)
        _accum(gath_v, outb_v, 1)
        _store(b1, 1)
        return 0

    lax.fori_loop(0, NB // 2, outer, 0)
    _drain(w_hbm, idx_v, gath_v, sem0, 0)


_mek = functools.partial(
    pl.kernel,
    out_type=jax.ShapeDtypeStruct((ROWS, D), jnp.float32),
    mesh=plsc.VectorSubcoreMesh(core_axis_name="c", subcore_axis_name="s"),
    scratch_types=[
        pltpu.VMEM((L, RPW), jnp.int32),
        pltpu.VMEM((2, L, B, D), jnp.float32),
        pltpu.VMEM((2, B, D), jnp.float32),
        pltpu.SemaphoreType.DMA,
        pltpu.SemaphoreType.DMA,
    ],
)(_body)


@jax.jit
def kernel(x, weight):
    x = x.astype(jnp.int32)
    w_flat = weight.reshape(L * NT, D)
    out = _mek(x, w_flat)
    return out.reshape(N, S, D)


# speedup vs baseline: 1.7503x; 1.7503x over previous
import functools

import jax
import jax.numpy as jnp
from jax import lax
from jax.experimental import pallas as pl
from jax.experimental.pallas import tpu as pltpu
from jax.experimental.pallas import tpu_sc as plsc

L = 8
NT = 1024
D = 1024
N = 4
S = 2048
ROWS = N * S
NC = 2
NS = 16
NW = NC * NS
RPW = ROWS // NW
B = 4
NB = RPW // B
LANES = 16
UNROLL = 4


def _fire(w_hbm, idx_v, gath_v, sem, b, buf):
    for l in range(L):
        pltpu.async_copy(
            w_hbm.at[idx_v.at[l, pl.ds(b * B, B)]],
            gath_v.at[buf, l],
            sem,
        )


def _drain(w_hbm, idx_v, gath_v, sem, buf):
    for l in range(L):
        pltpu.make_async_copy(
            w_hbm.at[idx_v.at[l, pl.ds(0, B)]],
            gath_v.at[buf, l],
            sem,
        ).wait()


def _accum(gath_v, outb_v, buf):
    for j in range(B):
        def cbody(c, _, j=j):
            o = pl.ds(pl.multiple_of(c * LANES, LANES), LANES)
            acc = gath_v[buf, 0, j, o]
            for l in range(1, L):
                acc = acc + gath_v[buf, l, j, o]
            outb_v[buf, j, o] = acc
            return 0
        lax.fori_loop(0, D // LANES, cbody, 0)


def _body(x_hbm, w_hbm, out_hbm, idx_v, gath_v, outb_v, sem0, sem1):
    cid = lax.axis_index("c")
    sid = lax.axis_index("s")
    wid = sid * NC + cid
    n = wid // (S // RPW)
    s0 = (wid % (S // RPW)) * RPW
    row0 = wid * RPW

    for l in range(L):
        pltpu.sync_copy(x_hbm.at[n, l, pl.ds(s0, RPW)], idx_v.at[l])

    def off_body(i, _):
        o = pl.ds(pl.multiple_of(i * LANES, LANES), LANES)
        for l in range(1, L):
            idx_v[l, o] = idx_v[l, o] + (l * NT)
        return 0
    lax.fori_loop(0, RPW // LANES, off_body, 0)

    def _store(b, buf):
        pltpu.sync_copy(outb_v.at[buf], out_hbm.at[pl.ds(row0 + b * B, B)])

    _fire(w_hbm, idx_v, gath_v, sem0, 0, 0)

    def outer(bb, _):
        b0 = 2 * bb
        b1 = 2 * bb + 1
        _fire(w_hbm, idx_v, gath_v, sem1, b1, 1)
        _drain(w_hbm, idx_v, gath_v, sem0, 0)
        _accum(gath_v, outb_v, 0)
        _store(b0, 0)
        _fire(w_hbm, idx_v, gath_v, sem0, jnp.minimum(b1 + 1, NB - 1), 0)
        _drain(w_hbm, idx_v, gath_v, sem1, 1)
        _accum(gath_v, outb_v, 1)
        _store(b1, 1)
        return 0

    lax.fori_loop(0, NB // 2, outer, 0)
    _drain(w_hbm, idx_v, gath_v, sem0, 0)


_mek = functools.partial(
    pl.kernel,
    out_type=jax.ShapeDtypeStruct((ROWS, D), jnp.float32),
    mesh=plsc.VectorSubcoreMesh(core_axis_name="c", subcore_axis_name="s"),
    scratch_types=[
        pltpu.VMEM((L, RPW), jnp.int32),
        pltpu.VMEM((2, L, B, D), jnp.float32),
        pltpu.VMEM((2, B, D), jnp.float32),
        pltpu.SemaphoreType.DMA,
        pltpu.SemaphoreType.DMA,
    ],
)(_body)


@jax.jit
def kernel(x, weight):
    x = x.astype(jnp.int32)
    w_flat = weight.reshape(L * NT, D)
    out = _mek(x, w_flat)
    return out.reshape(N, S, D)
